# baseline (speedup 1.0000x reference)
import numpy as np

import jax
import jax.numpy as jnp
from jax.experimental import pallas as pl
from jax.experimental.pallas import tpu as pltpu

_CP = 128


def _round_up(x, m):
  return ((x + m - 1) // m) * m


def _body(dims, s2_plan, s3_plan, xv_ref, w1_ref, sh_ref, *rest):
  B, BS, OP = dims
  n2, n3 = len(s2_plan[0]) + len(s2_plan[1]), len(s3_plan)
  w2_refs = rest[:n2]
  w3_refs = rest[n2:n2 + n3]
  o_ref, pb, qb = rest[n2 + n3:]
  sh1 = sh_ref[0:1, :]
  sh2 = sh_ref[1:2, :]
  sh3 = sh_ref[2:3, :]

  for b in range(B):
    xvb = xv_ref[b]
    x2b = jnp.concatenate([xvb[0:BS], xvb[1:BS + 1]], axis=1)

    for u in range(4):
      y = jnp.dot(x2b, w1_ref[:, 2 * _CP * u:2 * _CP * (u + 1)],
                  preferred_element_type=jnp.float32)
      p = jnp.maximum(
          jnp.maximum(y[:, 0:_CP], y[:, _CP:2 * _CP]) + sh1,
          0.0).astype(jnp.bfloat16)
      pb[0:BS, _CP * u:_CP * (u + 1)] = p

    wi = 0
    for u in range(2):
      acc = None
      for q, t0, t1 in s2_plan[u]:
        d = jnp.dot(pb[q:q + BS, _CP * t0:_CP * t1], w2_refs[wi][...],
                    preferred_element_type=jnp.float32)
        acc = d if acc is None else acc + d
        wi += 1
      qv = jnp.maximum(
          jnp.maximum(acc[:, 0:_CP], acc[:, _CP:2 * _CP]) + sh2,
          0.0).astype(jnp.bfloat16)
      qb[0:BS, _CP * u:_CP * (u + 1)] = qv

    acc3 = None
    for wi3, (q, t0, t1) in enumerate(s3_plan):
      d = jnp.dot(qb[q:q + BS, _CP * t0:_CP * t1], w3_refs[wi3][...],
                  preferred_element_type=jnp.float32)
      acc3 = d if acc3 is None else acc3 + d
    fin = jnp.maximum(
        jnp.maximum(acc3[:, 0:_CP], acc3[:, _CP:2 * _CP]) + sh3, 0.0)
    ft = jnp.transpose(fin[0:OP], (1, 0))
    o_ref[b] = ft[0:o_ref.shape[1], 0:o_ref.shape[2]]


@jax.jit
def kernel(x_ncl, w1, w2, w3, shifts):
  N, c_in, L = x_ncl.shape
  KC = w1.shape[0]
  K1 = KC // c_in
  K2, K3 = w2.shape[0], w3.shape[0]

  L0 = L + 8
  L_out1 = L0 - K1 + 1
  L_p1 = L_out1 // 2
  L_out2 = L_p1 - K2 + 1
  L_p2 = L_out2 // 2
  L_out3 = L_p2 - K3 + 1
  L_p3 = L_out3 // 2

  OP = _round_up(L_p3, 8)
  BS = _round_up(OP + 24, 16)
  BSX = BS + 8

  xp = jnp.pad(x_ncl, ((0, 0), (0, 0), (4, K1 * BSX - 4 - L)))
  xb = xp.astype(jnp.bfloat16).reshape(N, c_in, BSX, K1)
  xv = jnp.transpose(xb, (0, 2, 1, 3)).reshape(N, BSX, KC)

  j = np.arange(2 * KC)
  b, c, d = j // KC, (j % KC) // K1, j % K1
  W1_np = []
  for o in range(K1):
    idx = K1 * b + d - o
    valid = (idx >= 0) & (idx < K1)
    rows = np.clip(c_in * idx + c, 0, KC - 1)
    W1_np.append((rows, valid))
  W1 = jnp.concatenate(
      [jnp.where(jnp.asarray(v)[:, None], w1[jnp.asarray(r)], 0.0)
       for r, v in W1_np], axis=1).astype(jnp.bfloat16)

  def _shift_w(w, K, q, t0, t1, n_ph, stride):
    wh = w.reshape(K * _CP, _CP)
    jj = np.arange(_CP * (t1 - t0))
    a, cc = t0 + jj // _CP, jj % _CP
    cols = []
    for h in range(n_ph):
      k = stride * q + a - h
      valid = (k >= 0) & (k < K)
      rows = np.clip(k * _CP + cc, 0, K * _CP - 1)
      cols.append(jnp.where(jnp.asarray(valid)[:, None],
                            wh[jnp.asarray(rows)], 0.0))
    return jnp.concatenate(cols, axis=1).astype(jnp.bfloat16)

  s2_plan = [[(0, 0, 4), (1, 0, 4), (2, 0, 1)],
             [(0, 2, 4), (1, 0, 4), (2, 0, 3)]]
  s3_plan = [(0, 0, 2), (1, 0, 2), (2, 0, 1)]
  def _pair_w(w, K, q, t0, t1, u):
    wh = w.reshape(K * _CP, _CP)
    jj = np.arange(_CP * (t1 - t0))
    a, cc = t0 + jj // _CP, jj % _CP
    cols = []
    for h in range(2):
      k = 4 * q + a - (2 * u + h)
      valid = (k >= 0) & (k < K)
      rows = np.clip(k * _CP + cc, 0, K * _CP - 1)
      cols.append(jnp.where(jnp.asarray(valid)[:, None],
                            wh[jnp.asarray(rows)], 0.0))
    return jnp.concatenate(cols, axis=1).astype(jnp.bfloat16)

  W2s = [_pair_w(w2, K2, q, t0, t1, u)
         for u in range(2) for (q, t0, t1) in s2_plan[u]]
  W3s = [_shift_w(w3, K3, q, t0, t1, 2, 2) for (q, t0, t1) in s3_plan]

  B = 32 if N % 32 == 0 else 1
  wspecs = [pl.BlockSpec(w.shape, lambda n: (0, 0)) for w in W2s + W3s]
  out = pl.pallas_call(
      lambda *refs: _body((B, BS, OP), s2_plan, s3_plan, *refs),
      out_shape=jax.ShapeDtypeStruct((N, 50, L_p3), jnp.float32),
      grid=(N // B,),
      in_specs=[
          pl.BlockSpec((B, BSX, KC), lambda n: (n, 0, 0)),
          pl.BlockSpec(W1.shape, lambda n: (0, 0)),
          pl.BlockSpec(shifts.shape, lambda n: (0, 0)),
      ] + wspecs,
      out_specs=pl.BlockSpec((B, 50, L_p3), lambda n: (n, 0, 0)),
      scratch_shapes=[
          pltpu.VMEM((BS + 8, 4 * _CP), jnp.bfloat16),
          pltpu.VMEM((BS + 8, 2 * _CP), jnp.bfloat16),
      ],
      compiler_params=pltpu.CompilerParams(
          dimension_semantics=("parallel",)),
  )(xv, W1, shifts, *W2s, *W3s)

  return out


# speedup vs baseline: 1.1541x; 1.1541x over previous
import numpy as np

import jax
import jax.numpy as jnp
from jax.experimental import pallas as pl
from jax.experimental.pallas import tpu as pltpu

_CP = 128


def _round_up(x, m):
  return ((x + m - 1) // m) * m


def _body(dims, s2_plan, s3_plan, xv_ref, w1_ref, sh_ref, *rest):
  B, BS, OP = dims
  n2, n3 = len(s2_plan[0]) + len(s2_plan[1]), len(s3_plan)
  w2_refs = rest[:n2]
  w3_refs = rest[n2:n2 + n3]
  o_ref, pb, qb = rest[n2 + n3:]
  sh1 = sh_ref[0:1, :]
  sh2 = sh_ref[1:2, :]
  sh3 = sh_ref[2:3, :]

  for b in range(B):
    xvb = xv_ref[b]
    x2b = jnp.concatenate([xvb[0:BS], xvb[1:BS + 1]], axis=1)

    for u in range(4):
      y = jnp.dot(x2b, w1_ref[:, 2 * _CP * u:2 * _CP * (u + 1)],
                  preferred_element_type=jnp.float32)
      p = jnp.maximum(
          jnp.maximum(y[:, 0:_CP], y[:, _CP:2 * _CP]) + sh1,
          0.0).astype(jnp.bfloat16)
      pb[0:BS, _CP * u:_CP * (u + 1)] = p

    wi = 0
    for u in range(2):
      acc = None
      for q, t0, t1 in s2_plan[u]:
        d = jnp.dot(pb[q:q + BS, _CP * t0:_CP * t1], w2_refs[wi][...],
                    preferred_element_type=jnp.float32)
        acc = d if acc is None else acc + d
        wi += 1
      qv = jnp.maximum(
          jnp.maximum(acc[:, 0:_CP], acc[:, _CP:2 * _CP]) + sh2,
          0.0).astype(jnp.bfloat16)
      qb[0:BS, _CP * u:_CP * (u + 1)] = qv

    acc3 = None
    for wi3, (q, t0, t1) in enumerate(s3_plan):
      d = jnp.dot(qb[q:q + BS, _CP * t0:_CP * t1], w3_refs[wi3][...],
                  preferred_element_type=jnp.float32)
      acc3 = d if acc3 is None else acc3 + d
    fin = jnp.maximum(
        jnp.maximum(acc3[:, 0:_CP], acc3[:, _CP:2 * _CP]) + sh3, 0.0)
    ft = jnp.transpose(fin[0:OP], (1, 0))
    o_ref[b] = ft[0:o_ref.shape[1], 0:o_ref.shape[2]]


@jax.jit
def kernel(x_ncl, w1, w2, w3, shifts):
  N, c_in, L = x_ncl.shape
  KC = w1.shape[0]
  K1 = KC // c_in
  K2, K3 = w2.shape[0], w3.shape[0]

  L0 = L + 8
  L_out1 = L0 - K1 + 1
  L_p1 = L_out1 // 2
  L_out2 = L_p1 - K2 + 1
  L_p2 = L_out2 // 2
  L_out3 = L_p2 - K3 + 1
  L_p3 = L_out3 // 2

  OP = _round_up(L_p3, 8)
  BS = OP + 24
  BSX = BS + 8

  xp = jnp.pad(x_ncl, ((0, 0), (0, 0), (4, K1 * BSX - 4 - L)))
  xb = xp.astype(jnp.bfloat16).reshape(N, c_in, BSX, K1)
  xv = jnp.transpose(xb, (0, 2, 1, 3)).reshape(N, BSX, KC)

  j = np.arange(2 * KC)
  b, c, d = j // KC, (j % KC) // K1, j % K1
  W1_np = []
  for o in range(K1):
    idx = K1 * b + d - o
    valid = (idx >= 0) & (idx < K1)
    rows = np.clip(c_in * idx + c, 0, KC - 1)
    W1_np.append((rows, valid))
  W1 = jnp.concatenate(
      [jnp.where(jnp.asarray(v)[:, None], w1[jnp.asarray(r)], 0.0)
       for r, v in W1_np], axis=1).astype(jnp.bfloat16)

  def _shift_w(w, K, q, t0, t1, n_ph, stride):
    wh = w.reshape(K * _CP, _CP)
    jj = np.arange(_CP * (t1 - t0))
    a, cc = t0 + jj // _CP, jj % _CP
    cols = []
    for h in range(n_ph):
      k = stride * q + a - h
      valid = (k >= 0) & (k < K)
      rows = np.clip(k * _CP + cc, 0, K * _CP - 1)
      cols.append(jnp.where(jnp.asarray(valid)[:, None],
                            wh[jnp.asarray(rows)], 0.0))
    return jnp.concatenate(cols, axis=1).astype(jnp.bfloat16)

  s2_plan = [[(0, 0, 4), (1, 0, 4), (2, 0, 1)],
             [(0, 2, 4), (1, 0, 4), (2, 0, 3)]]
  s3_plan = [(0, 0, 2), (1, 0, 2), (2, 0, 1)]
  def _pair_w(w, K, q, t0, t1, u):
    wh = w.reshape(K * _CP, _CP)
    jj = np.arange(_CP * (t1 - t0))
    a, cc = t0 + jj // _CP, jj % _CP
    cols = []
    for h in range(2):
      k = 4 * q + a - (2 * u + h)
      valid = (k >= 0) & (k < K)
      rows = np.clip(k * _CP + cc, 0, K * _CP - 1)
      cols.append(jnp.where(jnp.asarray(valid)[:, None],
                            wh[jnp.asarray(rows)], 0.0))
    return jnp.concatenate(cols, axis=1).astype(jnp.bfloat16)

  W2s = [_pair_w(w2, K2, q, t0, t1, u)
         for u in range(2) for (q, t0, t1) in s2_plan[u]]
  W3s = [_shift_w(w3, K3, q, t0, t1, 2, 2) for (q, t0, t1) in s3_plan]

  B = 16 if N % 16 == 0 else 1
  wspecs = [pl.BlockSpec(w.shape, lambda n: (0, 0)) for w in W2s + W3s]
  out = pl.pallas_call(
      lambda *refs: _body((B, BS, OP), s2_plan, s3_plan, *refs),
      out_shape=jax.ShapeDtypeStruct((N, 50, L_p3), jnp.float32),
      grid=(N // B,),
      in_specs=[
          pl.BlockSpec((B, BSX, KC), lambda n: (n, 0, 0)),
          pl.BlockSpec(W1.shape, lambda n: (0, 0)),
          pl.BlockSpec(shifts.shape, lambda n: (0, 0)),
      ] + wspecs,
      out_specs=pl.BlockSpec((B, 50, L_p3), lambda n: (n, 0, 0)),
      scratch_shapes=[
          pltpu.VMEM((BS + 8, 4 * _CP), jnp.bfloat16),
          pltpu.VMEM((BS + 8, 2 * _CP), jnp.bfloat16),
      ],
      compiler_params=pltpu.CompilerParams(
          dimension_semantics=("parallel",)),
  )(xv, W1, shifts, *W2s, *W3s)

  return out


# speedup vs baseline: 1.1741x; 1.0173x over previous
import numpy as np

import jax
import jax.numpy as jnp
from jax.experimental import pallas as pl
from jax.experimental.pallas import tpu as pltpu

_CP = 128


def _round_up(x, m):
  return ((x + m - 1) // m) * m


def _body(dims, s2_plan, s3_plan, xv_ref, w1_ref, sh_ref, *rest):
  B, BS, OP = dims
  n2, n3 = len(s2_plan[0]) + len(s2_plan[1]), len(s3_plan)
  w2_refs = rest[:n2]
  w3_refs = rest[n2:n2 + n3]
  o_ref, pb, qb = rest[n2 + n3:]
  sh1 = sh_ref[0:1, :]
  sh2 = sh_ref[1:2, :]
  sh3 = sh_ref[2:3, :]

  for b in range(B):
    xvb = xv_ref[b]
    x2b = jnp.concatenate([xvb[0:BS], xvb[1:BS + 1]], axis=1)

    for u in range(4):
      y = jnp.dot(x2b, w1_ref[:, 2 * _CP * u:2 * _CP * (u + 1)],
                  preferred_element_type=jnp.float32)
      p = jnp.maximum(
          jnp.maximum(y[:, 0:_CP], y[:, _CP:2 * _CP]) + sh1,
          0.0).astype(jnp.bfloat16)
      pb[0:BS, _CP * u:_CP * (u + 1)] = p

    wi = 0
    for u in range(2):
      acc = None
      for q, t0, t1 in s2_plan[u]:
        d = jnp.dot(pb[q:q + BS, _CP * t0:_CP * t1], w2_refs[wi][...],
                    preferred_element_type=jnp.float32)
        acc = d if acc is None else acc + d
        wi += 1
      qv = jnp.maximum(
          jnp.maximum(acc[:, 0:_CP], acc[:, _CP:2 * _CP]) + sh2,
          0.0).astype(jnp.bfloat16)
      qb[0:BS, _CP * u:_CP * (u + 1)] = qv

    acc3 = None
    for wi3, (q, t0, t1) in enumerate(s3_plan):
      d = jnp.dot(qb[q:q + BS, _CP * t0:_CP * t1], w3_refs[wi3][...],
                  preferred_element_type=jnp.float32)
      acc3 = d if acc3 is None else acc3 + d
    fin = jnp.maximum(
        jnp.maximum(acc3[:, 0:_CP], acc3[:, _CP:2 * _CP]) + sh3, 0.0)
    ft = jnp.transpose(fin[0:OP], (1, 0))
    o_ref[b] = ft[0:o_ref.shape[1], 0:o_ref.shape[2]]


@jax.jit
def kernel(x_ncl, w1, w2, w3, shifts):
  N, c_in, L = x_ncl.shape
  KC = w1.shape[0]
  K1 = KC // c_in
  K2, K3 = w2.shape[0], w3.shape[0]

  L0 = L + 8
  L_out1 = L0 - K1 + 1
  L_p1 = L_out1 // 2
  L_out2 = L_p1 - K2 + 1
  L_p2 = L_out2 // 2
  L_out3 = L_p2 - K3 + 1
  L_p3 = L_out3 // 2

  OP = _round_up(L_p3, 8)
  BS = _round_up(OP + 24, 16)
  BSX = BS + 8

  xp = jnp.pad(x_ncl, ((0, 0), (0, 0), (4, K1 * BSX - 4 - L)))
  xb = xp.astype(jnp.bfloat16).reshape(N, c_in, BSX, K1)
  xv = jnp.transpose(xb, (0, 2, 1, 3)).reshape(N, BSX, KC)

  j = np.arange(2 * KC)
  b, c, d = j // KC, (j % KC) // K1, j % K1
  W1_np = []
  for o in range(K1):
    idx = K1 * b + d - o
    valid = (idx >= 0) & (idx < K1)
    rows = np.clip(c_in * idx + c, 0, KC - 1)
    W1_np.append((rows, valid))
  W1 = jnp.concatenate(
      [jnp.where(jnp.asarray(v)[:, None], w1[jnp.asarray(r)], 0.0)
       for r, v in W1_np], axis=1).astype(jnp.bfloat16)

  def _shift_w(w, K, q, t0, t1, n_ph, stride):
    wh = w.reshape(K * _CP, _CP)
    jj = np.arange(_CP * (t1 - t0))
    a, cc = t0 + jj // _CP, jj % _CP
    cols = []
    for h in range(n_ph):
      k = stride * q + a - h
      valid = (k >= 0) & (k < K)
      rows = np.clip(k * _CP + cc, 0, K * _CP - 1)
      cols.append(jnp.where(jnp.asarray(valid)[:, None],
                            wh[jnp.asarray(rows)], 0.0))
    return jnp.concatenate(cols, axis=1).astype(jnp.bfloat16)

  s2_plan = [[(0, 0, 4), (1, 0, 4), (2, 0, 1)],
             [(0, 2, 4), (1, 0, 4), (2, 0, 3)]]
  s3_plan = [(0, 0, 2), (1, 0, 2), (2, 0, 1)]
  def _pair_w(w, K, q, t0, t1, u):
    wh = w.reshape(K * _CP, _CP)
    jj = np.arange(_CP * (t1 - t0))
    a, cc = t0 + jj // _CP, jj % _CP
    cols = []
    for h in range(2):
      k = 4 * q + a - (2 * u + h)
      valid = (k >= 0) & (k < K)
      rows = np.clip(k * _CP + cc, 0, K * _CP - 1)
      cols.append(jnp.where(jnp.asarray(valid)[:, None],
                            wh[jnp.asarray(rows)], 0.0))
    return jnp.concatenate(cols, axis=1).astype(jnp.bfloat16)

  W2s = [_pair_w(w2, K2, q, t0, t1, u)
         for u in range(2) for (q, t0, t1) in s2_plan[u]]
  W3s = [_shift_w(w3, K3, q, t0, t1, 2, 2) for (q, t0, t1) in s3_plan]

  B = 16 if N % 16 == 0 else 1
  wspecs = [pl.BlockSpec(w.shape, lambda n: (0, 0)) for w in W2s + W3s]
  out = pl.pallas_call(
      lambda *refs: _body((B, BS, OP), s2_plan, s3_plan, *refs),
      out_shape=jax.ShapeDtypeStruct((N, 50, L_p3), jnp.float32),
      grid=(N // B,),
      in_specs=[
          pl.BlockSpec((B, BSX, KC), lambda n: (n, 0, 0)),
          pl.BlockSpec(W1.shape, lambda n: (0, 0)),
          pl.BlockSpec(shifts.shape, lambda n: (0, 0)),
      ] + wspecs,
      out_specs=pl.BlockSpec((B, 50, L_p3), lambda n: (n, 0, 0)),
      scratch_shapes=[
          pltpu.VMEM((BS + 8, 4 * _CP), jnp.bfloat16),
          pltpu.VMEM((BS + 8, 2 * _CP), jnp.bfloat16),
      ],
      compiler_params=pltpu.CompilerParams(
          dimension_semantics=("parallel",)),
  )(xv, W1, shifts, *W2s, *W3s)

  return out
